# baseline (speedup 1.0000x reference)
import functools

import jax
import jax.numpy as jnp
from jax import lax
from jax.experimental import pallas as pl
from jax.experimental.pallas import tpu as pltpu

_B, _S, _NX, _NE, _K = 4, 2048, 4096, 64, 8
_S_BLK = 512
_N_SBLK = _S // _S_BLK


def _router_kernel(xa_ref, xb_ref, w_ref, b_ref, vals_ref, idx_ref, acc_ref):
    bi = pl.program_id(0)
    j = pl.program_id(1)

    @pl.when(jnp.logical_and(bi == 0, j == 0))
    def _init():
        acc_ref[...] = jnp.zeros_like(acc_ref)

    ones = jnp.ones((1, _S_BLK), jnp.float32)
    partial = (jnp.dot(ones, xa_ref[0], precision=lax.Precision.DEFAULT,
                       preferred_element_type=jnp.float32)
               + jnp.dot(ones, xb_ref[0], precision=lax.Precision.DEFAULT,
                         preferred_element_type=jnp.float32))
    acc_ref[pl.ds(bi, 1), :] += partial

    @pl.when(jnp.logical_and(bi == _B - 1, j == _N_SBLK // 2 - 1))
    def _finalize():
        xm = acc_ref[...] * (1.0 / _S)
        xh = xm.astype(jnp.bfloat16)
        xl = (xm - xh.astype(jnp.float32)).astype(jnp.bfloat16)
        wb = w_ref[...]
        scores = (jnp.dot(xh, wb, preferred_element_type=jnp.float32)
                  + jnp.dot(xl, wb, preferred_element_type=jnp.float32)
                  + b_ref[...])
        m = jnp.max(scores, axis=1, keepdims=True)
        e = jnp.exp(scores - m)
        p = e / jnp.sum(e, axis=1, keepdims=True)

        iota = lax.broadcasted_iota(jnp.int32, (_B, _NE), 1)
        s = p
        for k in range(_K):
            mk = jnp.max(s, axis=1, keepdims=True)
            ik = jnp.min(jnp.where(s == mk, iota, _NE),
                         axis=1, keepdims=True)
            vals_ref[:, k:k + 1] = mk
            idx_ref[:, k:k + 1] = ik
            s = jnp.where(iota == ik, -jnp.inf, s)


@jax.jit
def kernel(x, W, b):
    b2 = b.reshape(1, _NE)
    wb16 = W.astype(jnp.bfloat16)
    vals, idx = pl.pallas_call(
        _router_kernel,
        grid=(_B, _N_SBLK // 2),
        in_specs=[
            pl.BlockSpec((1, _S_BLK, _NX), lambda bi, j: (bi, j, 0)),
            pl.BlockSpec((1, _S_BLK, _NX),
                         lambda bi, j: (bi, j + _N_SBLK // 2, 0)),
            pl.BlockSpec((_NX, _NE), lambda bi, j: (0, 0)),
            pl.BlockSpec((1, _NE), lambda bi, j: (0, 0)),
        ],
        out_specs=[
            pl.BlockSpec((_B, _K), lambda bi, j: (0, 0)),
            pl.BlockSpec((_B, _K), lambda bi, j: (0, 0)),
        ],
        out_shape=[
            jax.ShapeDtypeStruct((_B, _K), jnp.float32),
            jax.ShapeDtypeStruct((_B, _K), jnp.int32),
        ],
        scratch_shapes=[pltpu.VMEM((_B, _NX), jnp.float32)],
        compiler_params=pltpu.CompilerParams(
            dimension_semantics=("arbitrary", "arbitrary"),
        ),
    )(x, x, wb16, b2)
    return vals, idx


# speedup vs baseline: 1.0514x; 1.0514x over previous
import functools

import jax
import jax.numpy as jnp
from jax import lax
from jax.experimental import pallas as pl
from jax.experimental.pallas import tpu as pltpu

_B, _S, _NX, _NE, _K = 4, 2048, 4096, 64, 8
_S_BLK = 512
_N_SBLK = _S // _S_BLK


def _router_kernel(x_ref, w_ref, b_ref, vals_ref, idx_ref, acc_ref):
    bi = pl.program_id(0)
    j = pl.program_id(1)

    @pl.when(jnp.logical_and(bi == 0, j == 0))
    def _init():
        acc_ref[...] = jnp.zeros_like(acc_ref)

    ones = jnp.ones((1, _S_BLK), jnp.float32)
    partial = jnp.dot(ones, x_ref[0], precision=lax.Precision.DEFAULT,
                      preferred_element_type=jnp.float32)
    acc_ref[pl.ds(bi, 1), :] += partial

    @pl.when(jnp.logical_and(bi == _B - 1, j == _N_SBLK - 1))
    def _finalize():
        xs = acc_ref[...]
        xh = xs.astype(jnp.bfloat16)
        xl = (xs - xh.astype(jnp.float32)).astype(jnp.bfloat16)
        stacked = jnp.concatenate([xh, xl], axis=0)
        r = jnp.dot(stacked, w_ref[...],
                    preferred_element_type=jnp.float32)
        scores = (r[:_B] + r[_B:]) * (1.0 / _S) + b_ref[...]
        m = jnp.max(scores, axis=1, keepdims=True)
        e = jnp.exp(scores - m)
        p = e / jnp.sum(e, axis=1, keepdims=True)

        iota = lax.broadcasted_iota(jnp.int32, (_B, _NE), 1)
        s = p
        for k in range(_K):
            mk = jnp.max(s, axis=1, keepdims=True)
            ik = jnp.min(jnp.where(s == mk, iota, _NE),
                         axis=1, keepdims=True)
            vals_ref[:, k:k + 1] = mk
            idx_ref[:, k:k + 1] = ik
            s = jnp.where(iota == ik, -jnp.inf, s)


@jax.jit
def kernel(x, W, b):
    b2 = b.reshape(1, _NE)
    wb16 = W.astype(jnp.bfloat16)
    vals, idx = pl.pallas_call(
        _router_kernel,
        grid=(_B, _N_SBLK),
        in_specs=[
            pl.BlockSpec((1, _S_BLK, _NX), lambda bi, j: (bi, j, 0)),
            pl.BlockSpec((_NX, _NE), lambda bi, j: (0, 0)),
            pl.BlockSpec((1, _NE), lambda bi, j: (0, 0)),
        ],
        out_specs=[
            pl.BlockSpec((_B, _K), lambda bi, j: (0, 0)),
            pl.BlockSpec((_B, _K), lambda bi, j: (0, 0)),
        ],
        out_shape=[
            jax.ShapeDtypeStruct((_B, _K), jnp.float32),
            jax.ShapeDtypeStruct((_B, _K), jnp.int32),
        ],
        scratch_shapes=[pltpu.VMEM((_B, _NX), jnp.float32)],
        compiler_params=pltpu.CompilerParams(
            dimension_semantics=("arbitrary", "arbitrary"),
        ),
    )(x, wb16, b2)
    return vals, idx
